# baseline (speedup 1.0000x reference)
import functools

import jax
import jax.numpy as jnp
from jax import lax
from jax.experimental import pallas as pl
from jax.experimental.pallas import tpu as pltpu
from jax.experimental.pallas import tpu_sc as plsc

N, E, D, G, C = 10000, 320000, 128, 16, 10

NC, NS, LANES = 2, 16, 16
NW = NC * NS
EPT = E // NW
K = 96
NCHUNK = 104
KT = 16
ACCN = 10240
RPT = ACCN // NS
ZR = 64
NZB = RPT // ZR
DEGN = 10240
DPT = DEGN // NS

R = 1000
NBLK = N // R

@functools.cache
def _mesh():
    return plsc.VectorSubcoreMesh(
        core_axis_name="c", subcore_axis_name="s",
        num_cores=NC, num_subcores=NS)


def _sc_deg_body(ei_hbm, ew_hbm, eit_hbm, ewt_hbm, out_hbm,
                 acc, eb0, eb1, eb2, wb0, wb1, wb2, ebt, wbt, zbuf,
                 semi0, semi1, semi2, sems0, sems1, sems2):
    cid = lax.axis_index("c")
    sid = lax.axis_index("s")
    wid = sid * NC + cid
    gbase = wid * NCHUNK

    EB = (eb0, eb1, eb2)
    WB = (wb0, wb1, wb2)
    SI = (semi0, semi1, semi2)
    SS = (sems0, sems1, sems2)

    def start_idx(c, e):
        @pl.when(jnp.asarray(c, jnp.int32) < NCHUNK)
        def _():
            pltpu.async_copy(ei_hbm.at[gbase + c], EB[e], SI[e])
            pltpu.async_copy(ew_hbm.at[gbase + c], WB[e], SI[e])

    def wait_idx(e):
        pltpu.make_async_copy(ei_hbm.at[0], EB[e], SI[e]).wait()
        pltpu.make_async_copy(ew_hbm.at[0], WB[e], SI[e]).wait()

    def start_scatter(e):
        pltpu.async_copy(WB[e], acc.at[EB[e].at[1]], SS[e], add=True)

    def wait_scatter(e):
        pltpu.make_async_copy(WB[e], acc.at[EB[e].at[1]], SS[e]).wait()

    def step(c, e, drain):
        wait_idx(e)
        if drain:
            wait_scatter((e + 1) % 3)
            start_idx(c + 1, (e + 1) % 3)
        start_scatter(e)

    for c in range(3):
        start_idx(c, c)

    def zz(i, _):
        zbuf[pl.ds(i * LANES, LANES)] = jnp.zeros((LANES,), jnp.float32)
        return 0
    lax.fori_loop(0, DPT // LANES, zz, 0)
    row0 = sid * DPT
    pltpu.sync_copy(zbuf, acc.at[pl.ds(row0, DPT)])
    plsc.subcore_barrier()

    for c in range(3):
        step(c, c % 3, drain=(c >= 2))

    def block(i, _):
        c0 = 3 + 3 * i
        for k in range(3):
            step(c0 + k, k % 3, drain=True)
        return 0
    lax.fori_loop(0, (NCHUNK - 3) // 3, block, 0)
    for c in range(3 + 3 * ((NCHUNK - 3) // 3), NCHUNK):
        step(c, c % 3, drain=True)

    wait_scatter((NCHUNK - 2) % 3)
    wait_scatter((NCHUNK - 1) % 3)

    pltpu.sync_copy(eit_hbm.at[wid], ebt)
    pltpu.sync_copy(ewt_hbm.at[wid], wbt)
    pltpu.sync_copy(wbt, acc.at[ebt.at[1]], add=True)
    plsc.subcore_barrier()

    pltpu.sync_copy(acc.at[pl.ds(row0, DPT)], zbuf)
    pltpu.sync_copy(zbuf, out_hbm.at[cid, pl.ds(row0, DPT)])


@functools.cache
def _sc_deg():
    return pl.kernel(
        _sc_deg_body,
        out_type=jax.ShapeDtypeStruct((NC, DEGN), jnp.float32),
        mesh=_mesh(),
        scratch_types=[
            pltpu.VMEM_SHARED((DEGN,), jnp.float32),
            pltpu.VMEM((2, K), jnp.int32),
            pltpu.VMEM((2, K), jnp.int32),
            pltpu.VMEM((2, K), jnp.int32),
            pltpu.VMEM((K,), jnp.float32),
            pltpu.VMEM((K,), jnp.float32),
            pltpu.VMEM((K,), jnp.float32),
            pltpu.VMEM((2, KT), jnp.int32),
            pltpu.VMEM((KT,), jnp.float32),
            pltpu.VMEM((DPT,), jnp.float32),
            pltpu.SemaphoreType.DMA,
            pltpu.SemaphoreType.DMA,
            pltpu.SemaphoreType.DMA,
            pltpu.SemaphoreType.DMA,
            pltpu.SemaphoreType.DMA,
            pltpu.SemaphoreType.DMA,
        ],
    )


def _sc_spmm_body(h_hbm, ei_hbm, ew_hbm, eit_hbm, ewt_hbm, out_hbm, acc,
                  eb0, eb1, eb2, eb3, wb0, wb1, wb2, wb3, ebt, wbt,
                  rows0, rows1, rows2, rowst,
                  semi0, semi1, semi2, semi3, semg0, semg1, semg2,
                  sems0, sems1, sems2):
    cid = lax.axis_index("c")
    sid = lax.axis_index("s")
    wid = sid * NC + cid
    gbase = wid * NCHUNK

    EB = (eb0, eb1, eb2, eb3)
    WB = (wb0, wb1, wb2, wb3)
    RW = (rows0, rows1, rows2)
    SI = (semi0, semi1, semi2, semi3)
    SG = (semg0, semg1, semg2)
    SS = (sems0, sems1, sems2)

    def start_idx(c, e):
        @pl.when(jnp.asarray(c, jnp.int32) < NCHUNK)
        def _():
            pltpu.async_copy(ei_hbm.at[gbase + c], EB[e], SI[e])
            pltpu.async_copy(ew_hbm.at[gbase + c], WB[e], SI[e])

    def wait_idx(e):
        pltpu.make_async_copy(ei_hbm.at[0], EB[e], SI[e]).wait()
        pltpu.make_async_copy(ew_hbm.at[0], WB[e], SI[e]).wait()

    def start_gather(e, b):
        pltpu.async_copy(h_hbm.at[EB[e].at[0]], RW[b], SG[b])

    def wait_gather(e, b):
        pltpu.make_async_copy(h_hbm.at[EB[e].at[0]], RW[b], SG[b]).wait()

    def start_scatter(e, b):
        pltpu.async_copy(RW[b], acc.at[EB[e].at[1]], SS[b], add=True)

    def wait_scatter(e, b):
        pltpu.make_async_copy(RW[b], acc.at[EB[e].at[1]], SS[b]).wait()

    def multiply_rows(rows, wb, ngrp):
        def grp(g, _):
            wvec = wb[pl.ds(g * LANES, LANES)]

            def medge(j, _):
                wv = lax.gather(
                    wvec, jnp.full((LANES, 1), j, jnp.int32),
                    lax.GatherDimensionNumbers(
                        offset_dims=(), collapsed_slice_dims=(0,),
                        start_index_map=(0,)),
                    (1,), mode=lax.GatherScatterMode.PROMISE_IN_BOUNDS)
                r = g * LANES + j
                for k in range(D // LANES):
                    sl = pl.ds(k * LANES, LANES)
                    rows[r, sl] = rows[r, sl] * wv
                return 0
            return lax.fori_loop(0, LANES, medge, 0)
        lax.fori_loop(0, ngrp, grp, 0)

    def multiply(e, b):
        multiply_rows(RW[b], WB[e], K // LANES)

    def stepF(c, k):
        e, b = k % 4, k % 3
        en, bn = (k + 1) % 4, (k + 1) % 3
        e2 = (k + 2) % 4
        wait_gather(e, b)
        wait_scatter(e2, bn)
        start_idx(c + 2, e2)
        wait_idx(en)
        start_gather(en, bn)
        multiply(e, b)
        start_scatter(e, b)

    start_idx(0, 0)
    start_idx(1, 1)

    def zr(i, _):
        def zc(k, _):
            rows0[i, pl.ds(k * LANES, LANES)] = jnp.zeros((LANES,), jnp.float32)
            return 0
        return lax.fori_loop(0, D // LANES, zc, 0)
    lax.fori_loop(0, ZR, zr, 0)
    row0 = sid * RPT
    for jz in range(NZB):
        pltpu.sync_copy(rows0.at[pl.ds(0, ZR)],
                        acc.at[pl.ds(row0 + jz * ZR, ZR)])
    plsc.subcore_barrier()

    wait_idx(0)
    start_gather(0, 0)
    for c in range(2):
        e, b = c % 4, c % 3
        en, bn = (c + 1) % 4, (c + 1) % 3
        wait_gather(e, b)
        start_idx(c + 2, (c + 2) % 4)
        wait_idx(en)
        start_gather(en, bn)
        multiply(e, b)
        start_scatter(e, b)

    def block(i, _):
        c0 = 2 + 12 * i
        for k in range(12):
            stepF(c0 + k, 2 + k)
        return 0
    lax.fori_loop(0, (NCHUNK - 6) // 12, block, 0)

    for c in range(2 + 12 * ((NCHUNK - 6) // 12), NCHUNK - 2):
        stepF(c, c)
    for c in (NCHUNK - 2, NCHUNK - 1):
        e, b = c % 4, c % 3
        en, bn = (c + 1) % 4, (c + 1) % 3
        wait_gather(e, b)
        wait_scatter((c + 2) % 4, bn)
        if c + 1 < NCHUNK:
            wait_idx(en)
            start_gather(en, bn)
        multiply(e, b)
        start_scatter(e, b)

    wait_scatter((NCHUNK - 2) % 4, (NCHUNK - 2) % 3)
    wait_scatter((NCHUNK - 1) % 4, (NCHUNK - 1) % 3)

    pltpu.sync_copy(eit_hbm.at[wid], ebt)
    pltpu.sync_copy(ewt_hbm.at[wid], wbt)
    pltpu.async_copy(h_hbm.at[ebt.at[0]], rowst, SG[0]).wait()
    multiply_rows(rowst, wbt, KT // LANES)
    pltpu.sync_copy(rowst, acc.at[ebt.at[1]], add=True)
    plsc.subcore_barrier()

    for jz in range(NZB):
        r0 = row0 + jz * ZR
        pltpu.sync_copy(acc.at[pl.ds(r0, ZR)], rows0.at[pl.ds(0, ZR)])
        pltpu.sync_copy(rows0.at[pl.ds(0, ZR)],
                        out_hbm.at[cid, pl.ds(r0, ZR)])


@functools.cache
def _sc_spmm():
    return pl.kernel(
        _sc_spmm_body,
        out_type=jax.ShapeDtypeStruct((NC, ACCN, D), jnp.float32),
        mesh=_mesh(),
        scratch_types=[
            pltpu.VMEM_SHARED((ACCN, D), jnp.float32),
            pltpu.VMEM((2, K), jnp.int32),
            pltpu.VMEM((2, K), jnp.int32),
            pltpu.VMEM((2, K), jnp.int32),
            pltpu.VMEM((2, K), jnp.int32),
            pltpu.VMEM((K,), jnp.float32),
            pltpu.VMEM((K,), jnp.float32),
            pltpu.VMEM((K,), jnp.float32),
            pltpu.VMEM((K,), jnp.float32),
            pltpu.VMEM((2, KT), jnp.int32),
            pltpu.VMEM((KT,), jnp.float32),
            pltpu.VMEM((K, D), jnp.float32),
            pltpu.VMEM((K, D), jnp.float32),
            pltpu.VMEM((K, D), jnp.float32),
            pltpu.VMEM((KT, D), jnp.float32),
            pltpu.SemaphoreType.DMA,
            pltpu.SemaphoreType.DMA,
            pltpu.SemaphoreType.DMA,
            pltpu.SemaphoreType.DMA,
            pltpu.SemaphoreType.DMA,
            pltpu.SemaphoreType.DMA,
            pltpu.SemaphoreType.DMA,
            pltpu.SemaphoreType.DMA,
            pltpu.SemaphoreType.DMA,
            pltpu.SemaphoreType.DMA,
        ],
    )


def _dis_block(deg_ref):
    dg = deg_ref[...]
    deg = 1.0 + dg[:, 0:1] + dg[:, 1:2]
    return jnp.broadcast_to(lax.rsqrt(deg), (R, D))


def _tc_first_body(deg_ref, x_ref, w_ref, h_ref):
    dis = _dis_block(deg_ref)
    h_ref[...] = lax.dot_general(
        dis * x_ref[...], w_ref[...],
        (((1,), (1,)), ((), ())), preferred_element_type=jnp.float32)


_tc_first = pl.pallas_call(
    _tc_first_body,
    grid=(NBLK,),
    in_specs=[
        pl.BlockSpec((R, 2), lambda i: (i, 0)),
        pl.BlockSpec((R, D), lambda i: (i, 0)),
        pl.BlockSpec((D, D), lambda i: (0, 0)),
    ],
    out_specs=pl.BlockSpec((R, D), lambda i: (i, 0)),
    out_shape=jax.ShapeDtypeStruct((N, D), jnp.float32),
)


def _tc_mid_body(sp_ref, h_ref, deg_ref, b_ref, w_ref, out_ref):
    s = sp_ref[0] + sp_ref[1]
    dis = _dis_block(deg_ref)
    a = jnp.maximum(dis * (s + h_ref[...]) + b_ref[...], 0.0)
    out_ref[...] = lax.dot_general(
        dis * a, w_ref[...],
        (((1,), (1,)), ((), ())), preferred_element_type=jnp.float32)


_tc_mid = pl.pallas_call(
    _tc_mid_body,
    grid=(NBLK,),
    in_specs=[
        pl.BlockSpec((NC, R, D), lambda i: (0, i, 0)),
        pl.BlockSpec((R, D), lambda i: (i, 0)),
        pl.BlockSpec((R, 2), lambda i: (i, 0)),
        pl.BlockSpec((1, D), lambda i: (0, 0)),
        pl.BlockSpec((D, D), lambda i: (0, 0)),
    ],
    out_specs=pl.BlockSpec((R, D), lambda i: (i, 0)),
    out_shape=jax.ShapeDtypeStruct((N, D), jnp.float32),
)


def _tc_final_body(sp_ref, h_ref, deg_ref, b_ref, batch_ref,
                   l1w_ref, l1b_ref, l2w_ref, l2b_ref,
                   out_ref, pool_acc, cnt_acc):
    i = pl.program_id(0)

    @pl.when(i == 0)
    def _():
        pool_acc[...] = jnp.zeros((G, D), jnp.float32)
        cnt_acc[...] = jnp.zeros((G, D), jnp.float32)

    s = sp_ref[0] + sp_ref[1]
    dis = _dis_block(deg_ref)
    h3 = jnp.maximum(dis * (s + h_ref[...]) + b_ref[...], 0.0)
    bt = batch_ref[0]
    iota = lax.broadcasted_iota(jnp.int32, (G, R), 0)
    m = (iota == bt).astype(jnp.float32)
    pool_acc[...] += lax.dot_general(
        m, h3, (((1,), (0,)), ((), ())), preferred_element_type=jnp.float32)
    cnt_acc[...] += jnp.broadcast_to(
        jnp.sum(m, axis=1, keepdims=True), (G, D))

    @pl.when(i == pl.num_programs(0) - 1)
    def _():
        pooled = pool_acc[...] / jnp.maximum(cnt_acc[...], 1.0)
        z = jnp.maximum(
            lax.dot_general(pooled, l1w_ref[...], (((1,), (1,)), ((), ())),
                            preferred_element_type=jnp.float32) + l1b_ref[...],
            0.0)
        z2 = lax.dot_general(z, l2w_ref[...], (((1,), (1,)), ((), ())),
                             preferred_element_type=jnp.float32) + l2b_ref[...]
        mx = jnp.max(z2, axis=1, keepdims=True)
        lse = jnp.log(jnp.sum(jnp.exp(z2 - mx), axis=1, keepdims=True)) + mx
        out_ref[...] = z2 - lse


_tc_final = pl.pallas_call(
    _tc_final_body,
    grid=(NBLK,),
    in_specs=[
        pl.BlockSpec((NC, R, D), lambda i: (0, i, 0)),
        pl.BlockSpec((R, D), lambda i: (i, 0)),
        pl.BlockSpec((R, 2), lambda i: (i, 0)),
        pl.BlockSpec((1, D), lambda i: (0, 0)),
        pl.BlockSpec((1, 1, R), lambda i: (i, 0, 0)),
        pl.BlockSpec((D, D), lambda i: (0, 0)),
        pl.BlockSpec((1, D), lambda i: (0, 0)),
        pl.BlockSpec((C, D), lambda i: (0, 0)),
        pl.BlockSpec((1, C), lambda i: (0, 0)),
    ],
    out_specs=pl.BlockSpec((G, C), lambda i: (0, 0)),
    out_shape=jax.ShapeDtypeStruct((G, C), jnp.float32),
    scratch_shapes=[
        pltpu.VMEM((G, D), jnp.float32),
        pltpu.VMEM((G, D), jnp.float32),
    ],
)


def kernel(x, edge_index, edge_weight, batch,
           W1, b1, W2, b2, W3, b3, lin1_W, lin1_b, lin2_W, lin2_b):
    src = edge_index[0]
    dst = edge_index[1]

    sc_deg = _sc_deg()
    sc_spmm = _sc_spmm()

    nmain = NW * NCHUNK * K
    ei = jnp.stack([src[:nmain].reshape(-1, K),
                    dst[:nmain].reshape(-1, K)], axis=1)
    ew = edge_weight[:nmain].reshape(-1, K)
    eit = jnp.stack([src[nmain:].reshape(NW, KT),
                     dst[nmain:].reshape(NW, KT)], axis=1)
    ewt = edge_weight[nmain:].reshape(NW, KT)

    degp = sc_deg(ei, ew, eit, ewt)
    deg_t = degp[:, :N].T
    batch3 = batch.reshape(NBLK, 1, R)

    h1 = _tc_first(deg_t, x, W1)
    s1 = sc_spmm(h1, ei, ew, eit, ewt)
    h2 = _tc_mid(s1, h1, deg_t, b1.reshape(1, D), W2)
    s2 = sc_spmm(h2, ei, ew, eit, ewt)
    h3 = _tc_mid(s2, h2, deg_t, b2.reshape(1, D), W3)
    s3 = sc_spmm(h3, ei, ew, eit, ewt)
    out = _tc_final(s3, h3, deg_t, b3.reshape(1, D), batch3,
                    lin1_W, lin1_b.reshape(1, D), lin2_W, lin2_b.reshape(1, C))
    return out


# speedup vs baseline: 1.0270x; 1.0270x over previous
import functools

import jax
import jax.numpy as jnp
from jax import lax
from jax.experimental import pallas as pl
from jax.experimental.pallas import tpu as pltpu
from jax.experimental.pallas import tpu_sc as plsc

N, E, D, G, C = 10000, 320000, 128, 16, 10

NC, NS, LANES = 2, 16, 16
NW = NC * NS
EPT = E // NW
K = 104
NCHUNK = 96
KT = 16
ACCN = 10240
RPT = ACCN // NS
ZR = 64
NZB = RPT // ZR
DEGN = 10240
DPT = DEGN // NS

R = 1000
NBLK = N // R

@functools.cache
def _mesh():
    return plsc.VectorSubcoreMesh(
        core_axis_name="c", subcore_axis_name="s",
        num_cores=NC, num_subcores=NS)


def _sc_deg_body(ei_hbm, ew_hbm, eit_hbm, ewt_hbm, out_hbm,
                 acc, eb0, eb1, eb2, wb0, wb1, wb2, ebt, wbt, zbuf,
                 semi0, semi1, semi2, sems0, sems1, sems2):
    cid = lax.axis_index("c")
    sid = lax.axis_index("s")
    wid = sid * NC + cid
    gbase = wid * NCHUNK

    EB = (eb0, eb1, eb2)
    WB = (wb0, wb1, wb2)
    SI = (semi0, semi1, semi2)
    SS = (sems0, sems1, sems2)

    def start_idx(c, e):
        @pl.when(jnp.asarray(c, jnp.int32) < NCHUNK)
        def _():
            pltpu.async_copy(ei_hbm.at[gbase + c], EB[e], SI[e])
            pltpu.async_copy(ew_hbm.at[gbase + c], WB[e], SI[e])

    def wait_idx(e):
        pltpu.make_async_copy(ei_hbm.at[0], EB[e], SI[e]).wait()
        pltpu.make_async_copy(ew_hbm.at[0], WB[e], SI[e]).wait()

    def start_scatter(e):
        pltpu.async_copy(WB[e], acc.at[EB[e].at[1]], SS[e], add=True)

    def wait_scatter(e):
        pltpu.make_async_copy(WB[e], acc.at[EB[e].at[1]], SS[e]).wait()

    def step(c, e, drain):
        wait_idx(e)
        if drain:
            wait_scatter((e + 1) % 3)
            start_idx(c + 1, (e + 1) % 3)
        start_scatter(e)

    for c in range(3):
        start_idx(c, c)

    def zz(i, _):
        zbuf[pl.ds(i * LANES, LANES)] = jnp.zeros((LANES,), jnp.float32)
        return 0
    lax.fori_loop(0, DPT // LANES, zz, 0)
    row0 = sid * DPT
    pltpu.sync_copy(zbuf, acc.at[pl.ds(row0, DPT)])
    plsc.subcore_barrier()

    for c in range(3):
        step(c, c % 3, drain=(c >= 2))

    def block(i, _):
        c0 = 3 + 3 * i
        for k in range(3):
            step(c0 + k, k % 3, drain=True)
        return 0
    lax.fori_loop(0, (NCHUNK - 3) // 3, block, 0)
    for c in range(3 + 3 * ((NCHUNK - 3) // 3), NCHUNK):
        step(c, c % 3, drain=True)

    wait_scatter((NCHUNK - 2) % 3)
    wait_scatter((NCHUNK - 1) % 3)

    pltpu.sync_copy(eit_hbm.at[wid], ebt)
    pltpu.sync_copy(ewt_hbm.at[wid], wbt)
    pltpu.sync_copy(wbt, acc.at[ebt.at[1]], add=True)
    plsc.subcore_barrier()

    pltpu.sync_copy(acc.at[pl.ds(row0, DPT)], zbuf)
    pltpu.sync_copy(zbuf, out_hbm.at[cid, pl.ds(row0, DPT)])


@functools.cache
def _sc_deg():
    return pl.kernel(
        _sc_deg_body,
        out_type=jax.ShapeDtypeStruct((NC, DEGN), jnp.float32),
        mesh=_mesh(),
        scratch_types=[
            pltpu.VMEM_SHARED((DEGN,), jnp.float32),
            pltpu.VMEM((2, K), jnp.int32),
            pltpu.VMEM((2, K), jnp.int32),
            pltpu.VMEM((2, K), jnp.int32),
            pltpu.VMEM((K,), jnp.float32),
            pltpu.VMEM((K,), jnp.float32),
            pltpu.VMEM((K,), jnp.float32),
            pltpu.VMEM((2, KT), jnp.int32),
            pltpu.VMEM((KT,), jnp.float32),
            pltpu.VMEM((DPT,), jnp.float32),
            pltpu.SemaphoreType.DMA,
            pltpu.SemaphoreType.DMA,
            pltpu.SemaphoreType.DMA,
            pltpu.SemaphoreType.DMA,
            pltpu.SemaphoreType.DMA,
            pltpu.SemaphoreType.DMA,
        ],
    )


def _sc_spmm_body(h_hbm, ei_hbm, ew_hbm, eit_hbm, ewt_hbm, out_hbm, acc,
                  eb0, eb1, eb2, eb3, wb0, wb1, wb2, wb3, ebt, wbt,
                  rows0, rows1, rows2, rowst,
                  semi0, semi1, semi2, semi3, semg0, semg1, semg2,
                  sems0, sems1, sems2):
    cid = lax.axis_index("c")
    sid = lax.axis_index("s")
    wid = sid * NC + cid
    gbase = wid * NCHUNK

    EB = (eb0, eb1, eb2, eb3)
    WB = (wb0, wb1, wb2, wb3)
    RW = (rows0, rows1, rows2)
    SI = (semi0, semi1, semi2, semi3)
    SG = (semg0, semg1, semg2)
    SS = (sems0, sems1, sems2)

    def start_idx(c, e):
        @pl.when(jnp.asarray(c, jnp.int32) < NCHUNK)
        def _():
            pltpu.async_copy(ei_hbm.at[gbase + c], EB[e], SI[e])
            pltpu.async_copy(ew_hbm.at[gbase + c], WB[e], SI[e])

    def wait_idx(e):
        pltpu.make_async_copy(ei_hbm.at[0], EB[e], SI[e]).wait()
        pltpu.make_async_copy(ew_hbm.at[0], WB[e], SI[e]).wait()

    def start_gather(e, b):
        pltpu.async_copy(h_hbm.at[EB[e].at[0]], RW[b], SG[b])

    def wait_gather(e, b):
        pltpu.make_async_copy(h_hbm.at[EB[e].at[0]], RW[b], SG[b]).wait()

    def start_scatter(e, b):
        pltpu.async_copy(RW[b], acc.at[EB[e].at[1]], SS[b], add=True)

    def wait_scatter(e, b):
        pltpu.make_async_copy(RW[b], acc.at[EB[e].at[1]], SS[b]).wait()

    def multiply_rows(rows, wb, ngrp):
        def grp(g, _):
            wvec = wb[pl.ds(g * LANES, LANES)]

            def medge(j, _):
                wv = lax.gather(
                    wvec, jnp.full((LANES, 1), j, jnp.int32),
                    lax.GatherDimensionNumbers(
                        offset_dims=(), collapsed_slice_dims=(0,),
                        start_index_map=(0,)),
                    (1,), mode=lax.GatherScatterMode.PROMISE_IN_BOUNDS)
                r = g * LANES + j
                for k in range(D // LANES):
                    sl = pl.ds(k * LANES, LANES)
                    rows[r, sl] = rows[r, sl] * wv
                return 0
            return lax.fori_loop(0, LANES, medge, 0)
        lax.fori_loop(0, ngrp, grp, 0)

    def multiply(e, b):
        multiply_rows(RW[b], WB[e], K // LANES)

    def stepF(c, k):
        e, b = k % 4, k % 3
        en, bn = (k + 1) % 4, (k + 1) % 3
        e2 = (k + 2) % 4
        wait_gather(e, b)
        wait_scatter(e2, bn)
        start_idx(c + 2, e2)
        wait_idx(en)
        start_gather(en, bn)
        multiply(e, b)
        start_scatter(e, b)

    start_idx(0, 0)
    start_idx(1, 1)

    def zr(i, _):
        def zc(k, _):
            rows0[i, pl.ds(k * LANES, LANES)] = jnp.zeros((LANES,), jnp.float32)
            return 0
        return lax.fori_loop(0, D // LANES, zc, 0)
    lax.fori_loop(0, ZR, zr, 0)
    row0 = sid * RPT
    for jz in range(NZB):
        pltpu.sync_copy(rows0.at[pl.ds(0, ZR)],
                        acc.at[pl.ds(row0 + jz * ZR, ZR)])
    plsc.subcore_barrier()

    wait_idx(0)
    start_gather(0, 0)
    for c in range(2):
        e, b = c % 4, c % 3
        en, bn = (c + 1) % 4, (c + 1) % 3
        wait_gather(e, b)
        start_idx(c + 2, (c + 2) % 4)
        wait_idx(en)
        start_gather(en, bn)
        multiply(e, b)
        start_scatter(e, b)

    def block(i, _):
        c0 = 2 + 12 * i
        for k in range(12):
            stepF(c0 + k, 2 + k)
        return 0
    lax.fori_loop(0, (NCHUNK - 6) // 12, block, 0)

    for c in range(2 + 12 * ((NCHUNK - 6) // 12), NCHUNK - 2):
        stepF(c, c)
    for c in (NCHUNK - 2, NCHUNK - 1):
        e, b = c % 4, c % 3
        en, bn = (c + 1) % 4, (c + 1) % 3
        wait_gather(e, b)
        wait_scatter((c + 2) % 4, bn)
        if c + 1 < NCHUNK:
            wait_idx(en)
            start_gather(en, bn)
        multiply(e, b)
        start_scatter(e, b)

    wait_scatter((NCHUNK - 2) % 4, (NCHUNK - 2) % 3)
    wait_scatter((NCHUNK - 1) % 4, (NCHUNK - 1) % 3)

    pltpu.sync_copy(eit_hbm.at[wid], ebt)
    pltpu.sync_copy(ewt_hbm.at[wid], wbt)
    pltpu.async_copy(h_hbm.at[ebt.at[0]], rowst, SG[0]).wait()
    multiply_rows(rowst, wbt, KT // LANES)
    pltpu.sync_copy(rowst, acc.at[ebt.at[1]], add=True)
    plsc.subcore_barrier()

    for jz in range(NZB):
        r0 = row0 + jz * ZR
        pltpu.sync_copy(acc.at[pl.ds(r0, ZR)], rows0.at[pl.ds(0, ZR)])
        pltpu.sync_copy(rows0.at[pl.ds(0, ZR)],
                        out_hbm.at[cid, pl.ds(r0, ZR)])


@functools.cache
def _sc_spmm():
    return pl.kernel(
        _sc_spmm_body,
        out_type=jax.ShapeDtypeStruct((NC, ACCN, D), jnp.float32),
        mesh=_mesh(),
        scratch_types=[
            pltpu.VMEM_SHARED((ACCN, D), jnp.float32),
            pltpu.VMEM((2, K), jnp.int32),
            pltpu.VMEM((2, K), jnp.int32),
            pltpu.VMEM((2, K), jnp.int32),
            pltpu.VMEM((2, K), jnp.int32),
            pltpu.VMEM((K,), jnp.float32),
            pltpu.VMEM((K,), jnp.float32),
            pltpu.VMEM((K,), jnp.float32),
            pltpu.VMEM((K,), jnp.float32),
            pltpu.VMEM((2, KT), jnp.int32),
            pltpu.VMEM((KT,), jnp.float32),
            pltpu.VMEM((K, D), jnp.float32),
            pltpu.VMEM((K, D), jnp.float32),
            pltpu.VMEM((K, D), jnp.float32),
            pltpu.VMEM((KT, D), jnp.float32),
            pltpu.SemaphoreType.DMA,
            pltpu.SemaphoreType.DMA,
            pltpu.SemaphoreType.DMA,
            pltpu.SemaphoreType.DMA,
            pltpu.SemaphoreType.DMA,
            pltpu.SemaphoreType.DMA,
            pltpu.SemaphoreType.DMA,
            pltpu.SemaphoreType.DMA,
            pltpu.SemaphoreType.DMA,
            pltpu.SemaphoreType.DMA,
        ],
    )


def _dis_block(deg_ref):
    dg = deg_ref[...]
    deg = 1.0 + dg[:, 0:1] + dg[:, 1:2]
    return jnp.broadcast_to(lax.rsqrt(deg), (R, D))


def _tc_first_body(deg_ref, x_ref, w_ref, h_ref):
    dis = _dis_block(deg_ref)
    h_ref[...] = lax.dot_general(
        dis * x_ref[...], w_ref[...],
        (((1,), (1,)), ((), ())), preferred_element_type=jnp.float32)


_tc_first = pl.pallas_call(
    _tc_first_body,
    grid=(NBLK,),
    in_specs=[
        pl.BlockSpec((R, 2), lambda i: (i, 0)),
        pl.BlockSpec((R, D), lambda i: (i, 0)),
        pl.BlockSpec((D, D), lambda i: (0, 0)),
    ],
    out_specs=pl.BlockSpec((R, D), lambda i: (i, 0)),
    out_shape=jax.ShapeDtypeStruct((N, D), jnp.float32),
)


def _tc_mid_body(sp_ref, h_ref, deg_ref, b_ref, w_ref, out_ref):
    s = sp_ref[0] + sp_ref[1]
    dis = _dis_block(deg_ref)
    a = jnp.maximum(dis * (s + h_ref[...]) + b_ref[...], 0.0)
    out_ref[...] = lax.dot_general(
        dis * a, w_ref[...],
        (((1,), (1,)), ((), ())), preferred_element_type=jnp.float32)


_tc_mid = pl.pallas_call(
    _tc_mid_body,
    grid=(NBLK,),
    in_specs=[
        pl.BlockSpec((NC, R, D), lambda i: (0, i, 0)),
        pl.BlockSpec((R, D), lambda i: (i, 0)),
        pl.BlockSpec((R, 2), lambda i: (i, 0)),
        pl.BlockSpec((1, D), lambda i: (0, 0)),
        pl.BlockSpec((D, D), lambda i: (0, 0)),
    ],
    out_specs=pl.BlockSpec((R, D), lambda i: (i, 0)),
    out_shape=jax.ShapeDtypeStruct((N, D), jnp.float32),
)


def _tc_final_body(sp_ref, h_ref, deg_ref, b_ref, batch_ref,
                   l1w_ref, l1b_ref, l2w_ref, l2b_ref,
                   out_ref, pool_acc, cnt_acc):
    i = pl.program_id(0)

    @pl.when(i == 0)
    def _():
        pool_acc[...] = jnp.zeros((G, D), jnp.float32)
        cnt_acc[...] = jnp.zeros((G, D), jnp.float32)

    s = sp_ref[0] + sp_ref[1]
    dis = _dis_block(deg_ref)
    h3 = jnp.maximum(dis * (s + h_ref[...]) + b_ref[...], 0.0)
    bt = batch_ref[0]
    iota = lax.broadcasted_iota(jnp.int32, (G, R), 0)
    m = (iota == bt).astype(jnp.float32)
    pool_acc[...] += lax.dot_general(
        m, h3, (((1,), (0,)), ((), ())), preferred_element_type=jnp.float32)
    cnt_acc[...] += jnp.broadcast_to(
        jnp.sum(m, axis=1, keepdims=True), (G, D))

    @pl.when(i == pl.num_programs(0) - 1)
    def _():
        pooled = pool_acc[...] / jnp.maximum(cnt_acc[...], 1.0)
        z = jnp.maximum(
            lax.dot_general(pooled, l1w_ref[...], (((1,), (1,)), ((), ())),
                            preferred_element_type=jnp.float32) + l1b_ref[...],
            0.0)
        z2 = lax.dot_general(z, l2w_ref[...], (((1,), (1,)), ((), ())),
                             preferred_element_type=jnp.float32) + l2b_ref[...]
        mx = jnp.max(z2, axis=1, keepdims=True)
        lse = jnp.log(jnp.sum(jnp.exp(z2 - mx), axis=1, keepdims=True)) + mx
        out_ref[...] = z2 - lse


_tc_final = pl.pallas_call(
    _tc_final_body,
    grid=(NBLK,),
    in_specs=[
        pl.BlockSpec((NC, R, D), lambda i: (0, i, 0)),
        pl.BlockSpec((R, D), lambda i: (i, 0)),
        pl.BlockSpec((R, 2), lambda i: (i, 0)),
        pl.BlockSpec((1, D), lambda i: (0, 0)),
        pl.BlockSpec((1, 1, R), lambda i: (i, 0, 0)),
        pl.BlockSpec((D, D), lambda i: (0, 0)),
        pl.BlockSpec((1, D), lambda i: (0, 0)),
        pl.BlockSpec((C, D), lambda i: (0, 0)),
        pl.BlockSpec((1, C), lambda i: (0, 0)),
    ],
    out_specs=pl.BlockSpec((G, C), lambda i: (0, 0)),
    out_shape=jax.ShapeDtypeStruct((G, C), jnp.float32),
    scratch_shapes=[
        pltpu.VMEM((G, D), jnp.float32),
        pltpu.VMEM((G, D), jnp.float32),
    ],
)


def kernel(x, edge_index, edge_weight, batch,
           W1, b1, W2, b2, W3, b3, lin1_W, lin1_b, lin2_W, lin2_b):
    src = edge_index[0]
    dst = edge_index[1]

    sc_deg = _sc_deg()
    sc_spmm = _sc_spmm()

    nmain = NW * NCHUNK * K
    ei = jnp.stack([src[:nmain].reshape(-1, K),
                    dst[:nmain].reshape(-1, K)], axis=1)
    ew = edge_weight[:nmain].reshape(-1, K)
    eit = jnp.stack([src[nmain:].reshape(NW, KT),
                     dst[nmain:].reshape(NW, KT)], axis=1)
    ewt = edge_weight[nmain:].reshape(NW, KT)

    degp = sc_deg(ei, ew, eit, ewt)
    deg_t = degp[:, :N].T
    batch3 = batch.reshape(NBLK, 1, R)

    h1 = _tc_first(deg_t, x, W1)
    s1 = sc_spmm(h1, ei, ew, eit, ewt)
    h2 = _tc_mid(s1, h1, deg_t, b1.reshape(1, D), W2)
    s2 = sc_spmm(h2, ei, ew, eit, ewt)
    h3 = _tc_mid(s2, h2, deg_t, b2.reshape(1, D), W3)
    s3 = sc_spmm(h3, ei, ew, eit, ewt)
    out = _tc_final(s3, h3, deg_t, b3.reshape(1, D), batch3,
                    lin1_W, lin1_b.reshape(1, D), lin2_W, lin2_b.reshape(1, C))
    return out
